# baseline (speedup 1.0000x reference)
import functools

import jax
import jax.numpy as jnp
from jax import lax
from jax.experimental import pallas as pl
from jax.experimental.pallas import tpu as pltpu
from jax.experimental.pallas import tpu_sc as plsc

B, T, D = 8, 3072, 1024
BT = B * T
_INFO = plsc.get_sparse_core_info()
NC, NS, L = _INFO.num_cores, _INFO.num_subcores, _INFO.num_lanes
NW = NC * NS
TW = T // NW
C = 16
NCH_B = TW // C
NCH = B * NCH_B
NBLK = TW // 32

_mesh = plsc.VectorSubcoreMesh(core_axis_name="c", subcore_axis_name="s")


@functools.partial(
    pl.kernel,
    mesh=_mesh,
    out_type=jax.ShapeDtypeStruct((BT, D), jnp.float32),
    scratch_types=[
        pltpu.VMEM((B * TW,), jnp.int32),
        pltpu.VMEM((32, D), jnp.float32),
        pltpu.VMEM((NBLK, D), jnp.float32),
        pltpu.VMEM((1, D), jnp.float32),
        pltpu.VMEM((1, D), jnp.float32),
        pltpu.VMEM((C, D), jnp.float32),
        pltpu.VMEM((C, D), jnp.float32),
        pltpu.VMEM((C, D), jnp.float32),
        pltpu.VMEM((C, D), jnp.float32),
        pltpu.VMEM((C, D), jnp.float32),
        pltpu.SemaphoreType.DMA,
        pltpu.SemaphoreType.DMA,
        pltpu.SemaphoreType.DMA,
        pltpu.SemaphoreType.DMA,
        pltpu.SemaphoreType.DMA,
    ],
)
def _embed_sc(idx_hbm, wte_hbm, rtab_hbm, ctab_hbm, htab_hbm, out_hbm,
              idx_w, col_v, rc_v, rbuf, hbuf, g0, g1, g2, o0, o1,
              gsem0, gsem1, gsem2, ssem0, ssem1):
    wid = lax.axis_index("s") * NC + lax.axis_index("c")
    toff = wid * TW

    for bi in range(B):
        pltpu.sync_copy(idx_hbm.at[pl.ds(bi * T + toff, TW)],
                        idx_w.at[pl.ds(bi * TW, TW)])
    pltpu.sync_copy(ctab_hbm, col_v)
    for k in range(NBLK):
        tblk = toff + 32 * k
        rblk = lax.rem(tblk, 1024) // 32
        hblk = tblk // 1024
        pltpu.sync_copy(rtab_hbm.at[pl.ds(rblk, 1)], rbuf)
        pltpu.sync_copy(htab_hbm.at[pl.ds(hblk, 1)], hbuf)

        def rcvec(v, carry, k=k):
            s = pl.ds(v * L, L)
            rc_v[k, s] = rbuf[0, s] + hbuf[0, s]
            return carry
        lax.fori_loop(0, D // L, rcvec, 0)

    def gather_desc(c, buf, sem):
        bi = c // NCH_B
        j = lax.rem(c, NCH_B)
        ioff = pl.multiple_of(bi * TW + j * C, 8)
        return pltpu.make_async_copy(
            wte_hbm.at[idx_w.at[pl.ds(ioff, C)]], buf, sem)

    def scatter_desc(c, buf, sem):
        bi = c // NCH_B
        j = lax.rem(c, NCH_B)
        return pltpu.make_async_copy(
            buf, out_hbm.at[pl.ds(bi * T + toff + j * C, C)], sem)

    def compute(c, g, o):
        j = lax.rem(c, NCH_B)
        kblk = j // (32 // C)
        colbase = lax.rem(j, 32 // C) * C
        nv = D // L

        @plsc.parallel_loop(0, C * nv, unroll=32)
        def _(v):
            r = v // nv
            s = pl.ds(lax.rem(v, nv) * L, L)
            o[r, s] = g[r, s] + col_v[colbase + r, s] + rc_v[kblk, s]

    gbufs = (g0, g1, g2)
    gsems = (gsem0, gsem1, gsem2)
    obufs = (o0, o1)
    ssems = (ssem0, ssem1)
    gather_desc(0, g0, gsem0).start()
    gather_desc(1, g1, gsem1).start()

    def six(i, carry):
        for p in range(6):
            c = 6 * i + p
            gb, gs = gbufs[p % 3], gsems[p % 3]
            ob, ss = obufs[p % 2], ssems[p % 2]
            ng, ngs = gbufs[(p + 2) % 3], gsems[(p + 2) % 3]
            gather_desc(c, gb, gs).wait()

            @pl.when(c + 2 < NCH)
            def _():
                gather_desc(c + 2, ng, ngs).start()

            @pl.when(c >= 2)
            def _():
                scatter_desc(c - 2, ob, ss).wait()

            compute(c, gb, ob)
            scatter_desc(c, ob, ss).start()
        return carry
    lax.fori_loop(0, NCH // 6, six, 0)
    scatter_desc(NCH - 2, obufs[(NCH - 2) % 2], ssems[(NCH - 2) % 2]).wait()
    scatter_desc(NCH - 1, obufs[(NCH - 1) % 2], ssems[(NCH - 1) % 2]).wait()


def kernel(idx, wte, row_emb, col_emb, chan_emb):
    b, t = idx.shape
    d = wte.shape[1]
    out = _embed_sc(idx.reshape(-1), wte, row_emb, col_emb, chan_emb)
    return out.reshape(b, t, d)


# speedup vs baseline: 1.0177x; 1.0177x over previous
import functools

import jax
import jax.numpy as jnp
from jax import lax
from jax.experimental import pallas as pl
from jax.experimental.pallas import tpu as pltpu
from jax.experimental.pallas import tpu_sc as plsc

B, T, D = 8, 3072, 1024
BT = B * T
_INFO = plsc.get_sparse_core_info()
NC, NS, L = _INFO.num_cores, _INFO.num_subcores, _INFO.num_lanes
NW = NC * NS
TW = T // NW
C = 16
NCH_B = TW // C
NCH = B * NCH_B
NBLK = TW // 32

_mesh = plsc.VectorSubcoreMesh(core_axis_name="c", subcore_axis_name="s")


@functools.partial(
    pl.kernel,
    mesh=_mesh,
    out_type=jax.ShapeDtypeStruct((BT, D), jnp.float32),
    scratch_types=[
        pltpu.VMEM((B * TW,), jnp.int32),
        pltpu.VMEM((32, D), jnp.float32),
        pltpu.VMEM((NBLK, D), jnp.float32),
        pltpu.VMEM((1, D), jnp.float32),
        pltpu.VMEM((1, D), jnp.float32),
        pltpu.VMEM((C, D), jnp.float32),
        pltpu.VMEM((C, D), jnp.float32),
        pltpu.VMEM((C, D), jnp.float32),
        pltpu.VMEM((C, D), jnp.float32),
        pltpu.VMEM((C, D), jnp.float32),
        pltpu.SemaphoreType.DMA,
        pltpu.SemaphoreType.DMA,
        pltpu.SemaphoreType.DMA,
        pltpu.SemaphoreType.DMA,
        pltpu.SemaphoreType.DMA,
    ],
)
def _embed_sc(idx_hbm, wte_hbm, rtab_hbm, ctab_hbm, htab_hbm, out_hbm,
              idx_w, col_v, rc_v, rbuf, hbuf, g0, g1, g2, o0, o1,
              gsem0, gsem1, gsem2, ssem0, ssem1):
    wid = lax.axis_index("s") * NC + lax.axis_index("c")
    toff = wid * TW

    for bi in range(B):
        pltpu.sync_copy(idx_hbm.at[pl.ds(bi * T + toff, TW)],
                        idx_w.at[pl.ds(bi * TW, TW)])
    pltpu.sync_copy(ctab_hbm, col_v)
    for k in range(NBLK):
        tblk = toff + 32 * k
        rblk = lax.rem(tblk, 1024) // 32
        hblk = tblk // 1024
        pltpu.sync_copy(rtab_hbm.at[pl.ds(rblk, 1)], rbuf)
        pltpu.sync_copy(htab_hbm.at[pl.ds(hblk, 1)], hbuf)

        def rcvec(v, carry, k=k):
            s = pl.ds(v * L, L)
            rc_v[k, s] = rbuf[0, s] + hbuf[0, s]
            return carry
        lax.fori_loop(0, D // L, rcvec, 0)

    def gather_desc(c, buf, sem):
        bi = c // NCH_B
        j = lax.rem(c, NCH_B)
        ioff = pl.multiple_of(bi * TW + j * C, 8)
        return pltpu.make_async_copy(
            wte_hbm.at[idx_w.at[pl.ds(ioff, C)]], buf, sem)

    def scatter_desc(c, buf, sem):
        bi = c // NCH_B
        j = lax.rem(c, NCH_B)
        return pltpu.make_async_copy(
            buf, out_hbm.at[pl.ds(bi * T + toff + j * C, C)], sem)

    def compute(c, g, o):
        j = lax.rem(c, NCH_B)
        kblk = j // (32 // C)
        colbase = lax.rem(j, 32 // C) * C
        nv = D // L

        @plsc.parallel_loop(0, C * nv, unroll=16)
        def _(v):
            r = v // nv
            s = pl.ds(lax.rem(v, nv) * L, L)
            o[r, s] = g[r, s] + col_v[colbase + r, s] + rc_v[kblk, s]

    gbufs = (g0, g1, g2)
    gsems = (gsem0, gsem1, gsem2)
    obufs = (o0, o1)
    ssems = (ssem0, ssem1)
    gather_desc(0, g0, gsem0).start()
    gather_desc(1, g1, gsem1).start()

    def six(i, carry):
        for p in range(6):
            c = 6 * i + p
            gb, gs = gbufs[p % 3], gsems[p % 3]
            ob, ss = obufs[p % 2], ssems[p % 2]
            ng, ngs = gbufs[(p + 2) % 3], gsems[(p + 2) % 3]
            gather_desc(c, gb, gs).wait()

            @pl.when(c + 2 < NCH)
            def _():
                gather_desc(c + 2, ng, ngs).start()

            @pl.when(c >= 2)
            def _():
                scatter_desc(c - 2, ob, ss).wait()

            compute(c, gb, ob)
            scatter_desc(c, ob, ss).start()
        return carry
    lax.fori_loop(0, NCH // 6, six, 0)
    scatter_desc(NCH - 2, obufs[(NCH - 2) % 2], ssems[(NCH - 2) % 2]).wait()
    scatter_desc(NCH - 1, obufs[(NCH - 1) % 2], ssems[(NCH - 1) % 2]).wait()


def kernel(idx, wte, row_emb, col_emb, chan_emb):
    b, t = idx.shape
    d = wte.shape[1]
    out = _embed_sc(idx.reshape(-1), wte, row_emb, col_emb, chan_emb)
    return out.reshape(b, t, d)


# speedup vs baseline: 1.0179x; 1.0003x over previous
import functools

import jax
import jax.numpy as jnp
from jax import lax
from jax.experimental import pallas as pl
from jax.experimental.pallas import tpu as pltpu
from jax.experimental.pallas import tpu_sc as plsc

B, T, D = 8, 3072, 1024
BT = B * T
_INFO = plsc.get_sparse_core_info()
NC, NS, L = _INFO.num_cores, _INFO.num_subcores, _INFO.num_lanes
NW = NC * NS
TW = T // NW
C = 16
NCH_B = TW // C
NCH = B * NCH_B
NBLK = TW // 32

_mesh = plsc.VectorSubcoreMesh(core_axis_name="c", subcore_axis_name="s")


@functools.partial(
    pl.kernel,
    mesh=_mesh,
    out_type=jax.ShapeDtypeStruct((BT, D), jnp.float32),
    scratch_types=[
        pltpu.VMEM((B * TW,), jnp.int32),
        pltpu.VMEM((32, D), jnp.float32),
        pltpu.VMEM((NBLK, D), jnp.float32),
        pltpu.VMEM((1, D), jnp.float32),
        pltpu.VMEM((1, D), jnp.float32),
        pltpu.VMEM((C, D), jnp.float32),
        pltpu.VMEM((C, D), jnp.float32),
        pltpu.VMEM((C, D), jnp.float32),
        pltpu.VMEM((C, D), jnp.float32),
        pltpu.VMEM((C, D), jnp.float32),
        pltpu.SemaphoreType.DMA,
        pltpu.SemaphoreType.DMA,
        pltpu.SemaphoreType.DMA,
        pltpu.SemaphoreType.DMA,
        pltpu.SemaphoreType.DMA,
    ],
)
def _embed_sc(idx_hbm, wte_hbm, rtab_hbm, ctab_hbm, htab_hbm, out_hbm,
              idx_w, col_v, rc_v, rbuf, hbuf, g0, g1, g2, o0, o1,
              gsem0, gsem1, gsem2, ssem0, ssem1):
    wid = lax.axis_index("s") * NC + lax.axis_index("c")
    toff = wid * TW

    for bi in range(B):
        pltpu.sync_copy(idx_hbm.at[pl.ds(bi * T + toff, TW)],
                        idx_w.at[pl.ds(bi * TW, TW)])

    def _early_gather(c, buf, sem):
        ioff = pl.multiple_of(c * C, 8)
        return pltpu.make_async_copy(
            wte_hbm.at[idx_w.at[pl.ds(ioff, C)]], buf, sem)

    _early_gather(0, g0, gsem0).start()
    _early_gather(1, g1, gsem1).start()
    pltpu.sync_copy(ctab_hbm, col_v)
    for k in range(NBLK):
        tblk = toff + 32 * k
        rblk = lax.rem(tblk, 1024) // 32
        hblk = tblk // 1024
        pltpu.sync_copy(rtab_hbm.at[pl.ds(rblk, 1)], rbuf)
        pltpu.sync_copy(htab_hbm.at[pl.ds(hblk, 1)], hbuf)

        def rcvec(v, carry, k=k):
            s = pl.ds(v * L, L)
            rc_v[k, s] = rbuf[0, s] + hbuf[0, s]
            return carry
        lax.fori_loop(0, D // L, rcvec, 0)

    def gather_desc(c, buf, sem):
        bi = c // NCH_B
        j = lax.rem(c, NCH_B)
        ioff = pl.multiple_of(bi * TW + j * C, 8)
        return pltpu.make_async_copy(
            wte_hbm.at[idx_w.at[pl.ds(ioff, C)]], buf, sem)

    def scatter_desc(c, buf, sem):
        bi = c // NCH_B
        j = lax.rem(c, NCH_B)
        return pltpu.make_async_copy(
            buf, out_hbm.at[pl.ds(bi * T + toff + j * C, C)], sem)

    def compute(c, g, o):
        j = lax.rem(c, NCH_B)
        kblk = j // (32 // C)
        colbase = lax.rem(j, 32 // C) * C
        nv = D // L

        @plsc.parallel_loop(0, C * nv, unroll=16)
        def _(v):
            r = v // nv
            s = pl.ds(lax.rem(v, nv) * L, L)
            o[r, s] = g[r, s] + col_v[colbase + r, s] + rc_v[kblk, s]

    gbufs = (g0, g1, g2)
    gsems = (gsem0, gsem1, gsem2)
    obufs = (o0, o1)
    ssems = (ssem0, ssem1)
    def six(i, carry):
        for p in range(6):
            c = 6 * i + p
            gb, gs = gbufs[p % 3], gsems[p % 3]
            ob, ss = obufs[p % 2], ssems[p % 2]
            ng, ngs = gbufs[(p + 2) % 3], gsems[(p + 2) % 3]
            gather_desc(c, gb, gs).wait()

            @pl.when(c + 2 < NCH)
            def _():
                gather_desc(c + 2, ng, ngs).start()

            @pl.when(c >= 2)
            def _():
                scatter_desc(c - 2, ob, ss).wait()

            compute(c, gb, ob)
            scatter_desc(c, ob, ss).start()
        return carry
    lax.fori_loop(0, NCH // 6, six, 0)
    scatter_desc(NCH - 2, obufs[(NCH - 2) % 2], ssems[(NCH - 2) % 2]).wait()
    scatter_desc(NCH - 1, obufs[(NCH - 1) % 2], ssems[(NCH - 1) % 2]).wait()


def kernel(idx, wte, row_emb, col_emb, chan_emb):
    b, t = idx.shape
    d = wte.shape[1]
    out = _embed_sc(idx.reshape(-1), wte, row_emb, col_emb, chan_emb)
    return out.reshape(b, t, d)
